# baseline (speedup 1.0000x reference)
import functools

import jax
import jax.numpy as jnp
from jax import lax
from jax.experimental import pallas as pl
from jax.experimental.pallas import tpu as pltpu
from jax.experimental.pallas import tpu_sc as plsc

_NC = 2
_NS = 16
_IDXL = 128
_SUPER_ROWS = 8
_CHUNK_ROWS = 2


def _sc_scatter(xcp, srow, sp64, drow, dp64, probsp, n2, d):
    rows_total = srow.shape[0]
    rows_per_tile = rows_total // _NS
    supers = rows_per_tile // _SUPER_ROWS
    chunk_edges = _CHUNK_ROWS * _IDXL
    n_per_tile = n2 // _NS
    dh = d // _NC

    mesh = plsc.VectorSubcoreMesh(core_axis_name="c", subcore_axis_name="s")

    @functools.partial(
        pl.kernel,
        out_type=jax.ShapeDtypeStruct((_NC, n2, d), jnp.float32),
        mesh=mesh,
        scratch_types=[
            pltpu.MemorySpace.VMEM_SHARED((n2, d), jnp.float32),
            pltpu.MemorySpace.VMEM_SHARED((n2, d), jnp.float32),
            pltpu.VMEM((_SUPER_ROWS, _IDXL), jnp.int32),
            pltpu.VMEM((_SUPER_ROWS, _IDXL), jnp.int32),
            pltpu.VMEM((_SUPER_ROWS, _IDXL), jnp.int32),
            pltpu.VMEM((_SUPER_ROWS, _IDXL), jnp.int32),
            pltpu.VMEM((_SUPER_ROWS, _IDXL), jnp.float32),
            pltpu.VMEM((chunk_edges, d), jnp.float32),
            pltpu.SemaphoreType.DMA,
        ],
    )
    def k(xcp_hbm, srow_hbm, sp64_hbm, drow_hbm, dp64_hbm, probs_hbm,
          out_hbm, xs, acc, sr, sp, dr, dp, pv, rows, sem):
        cid = lax.axis_index("c")
        sid = lax.axis_index("s")
        base_row = sid * rows_per_tile

        pltpu.sync_copy(xcp_hbm.at[cid, pl.ds(sid * n_per_tile, n_per_tile)],
                        xs.at[pl.ds(sid * n_per_tile, n_per_tile)])

        def zbody(r, carry):
            for g in range(d // 16):
                rows[r, pl.ds(g * 16, 16)] = jnp.zeros((16,), jnp.float32)
            return carry
        lax.fori_loop(0, 128, zbody, 0)
        for z, zr in ((0, 128), (128, 128), (256, 64)):
            pltpu.sync_copy(
                rows.at[pl.ds(0, zr)],
                acc.at[pl.ds(sid * n_per_tile + z, zr)])
        plsc.subcore_barrier()

        def super_body(s, carry):
            row0 = base_row + s * _SUPER_ROWS
            pltpu.sync_copy(srow_hbm.at[pl.ds(row0, _SUPER_ROWS)], sr)
            pltpu.sync_copy(sp64_hbm.at[pl.ds(row0, _SUPER_ROWS)], sp)
            pltpu.sync_copy(drow_hbm.at[pl.ds(row0, _SUPER_ROWS)], dr)
            pltpu.sync_copy(dp64_hbm.at[pl.ds(row0, _SUPER_ROWS)], dp)
            pltpu.sync_copy(probs_hbm.at[pl.ds(row0, _SUPER_ROWS)], pv)

            for c in range(_SUPER_ROWS // _CHUNK_ROWS):
                r0 = c * _CHUNK_ROWS
                cps = [
                    pltpu.async_copy(xs.at[sr.at[r0 + j]],
                                     rows.at[pl.ds(j * _IDXL, _IDXL)], sem)
                    for j in range(_CHUNK_ROWS)
                ]
                for cp in cps:
                    cp.wait()
                for j in range(_CHUNK_ROWS):
                    def scale_body(g, inner, j=j, r0=r0):
                        p16 = pv[r0 + j, pl.ds(g * 16, 16)]
                        so16 = sp[r0 + j, pl.ds(g * 16, 16)]
                        do16 = dp[r0 + j, pl.ds(g * 16, 16)]
                        for lane in range(16):
                            p = p16[lane]
                            so = so16[lane]
                            do = do16[lane]
                            rr = j * _IDXL + g * 16 + lane
                            vals = [
                                rows[rr, pl.ds(so + gg * 16, 16)] * p
                                for gg in range(dh // 16)
                            ]
                            zero = jnp.zeros((16,), jnp.float32)
                            for gg in range(dh // 16):
                                rows[rr, pl.ds(do + gg * 16, 16)] = vals[gg]
                                rows[rr, pl.ds((64 - do) + gg * 16, 16)] = zero
                        return inner
                    lax.fori_loop(0, _IDXL // 16, scale_body, 0)
                for j in range(_CHUNK_ROWS):
                    pltpu.sync_copy(rows.at[pl.ds(j * _IDXL, _IDXL)],
                                    acc.at[dr.at[r0 + j]], add=True)
            return carry
        lax.fori_loop(0, supers, super_body, 0)
        plsc.subcore_barrier()

        pltpu.sync_copy(acc.at[pl.ds(sid * n_per_tile, n_per_tile)],
                        out_hbm.at[cid, pl.ds(sid * n_per_tile, n_per_tile)])

    return k(xcp, srow, sp64, drow, dp64, probsp)


def _combine(x, partials, weight, slw):
    n = x.shape[0]

    def body(x_ref, p_ref, w_ref, s_ref, o_ref):
        s = s_ref[0, 0]
        agg = jnp.concatenate([p_ref[0][:n], p_ref[1][:n]], axis=1)
        o_ref[...] = jnp.clip(
            x_ref[...] * (1.0 + s) + agg * w_ref[...], 0.0, 1.0)

    return pl.pallas_call(
        body,
        out_shape=jax.ShapeDtypeStruct(x.shape, x.dtype),
    )(x, partials, weight, slw)


def kernel(x, edge_index, edge_probs, weight, self_loop_weight):
    n, d = x.shape
    dh = d // _NC
    e = edge_index.shape[1]
    gran = _NS * _IDXL * _SUPER_ROWS
    e_pad = ((e + gran - 1) // gran) * gran
    pad = e_pad - e

    src = jnp.concatenate([edge_index[0], jnp.zeros((pad,), jnp.int32)])
    dst = jnp.concatenate([edge_index[1], jnp.zeros((pad,), jnp.int32)])
    srow = (src >> 1).reshape(-1, _IDXL)
    sp64 = ((src & 1) << 6).reshape(-1, _IDXL)
    drow = (dst >> 1).reshape(-1, _IDXL)
    dp64 = ((dst & 1) << 6).reshape(-1, _IDXL)
    pr = jnp.concatenate(
        [edge_probs.astype(jnp.float32),
         jnp.zeros((pad,), jnp.float32)]).reshape(-1, _IDXL)

    n_pad = ((n + 2047) // 2048) * 2048
    n2 = n_pad // 2
    xpad = jnp.zeros((n_pad, d), jnp.float32).at[:n].set(
        x.astype(jnp.float32))
    xcp = jnp.stack([xpad[:, :dh].reshape(n2, d),
                     xpad[:, dh:].reshape(n2, d)])

    partials = _sc_scatter(xcp, srow, sp64, drow, dp64, pr, n2, d)
    partials = partials.reshape(_NC, n_pad, dh)
    w2 = weight.astype(jnp.float32).reshape(1, d)
    s2 = jnp.asarray(self_loop_weight, jnp.float32).reshape(1, 1)
    return _combine(x, partials, w2, s2)


# speedup vs baseline: 1.2797x; 1.2797x over previous
import functools

import jax
import jax.numpy as jnp
from jax import lax
from jax.experimental import pallas as pl
from jax.experimental.pallas import tpu as pltpu
from jax.experimental.pallas import tpu_sc as plsc

_NC = 2
_NS = 16
_NW = _NC * _NS
_CHUNK = 4096
_L = 16


def _sc_scatter(xt, srcp, dstp, probsp, n, d):
    e_pad = srcp.shape[0]
    chunks = e_pad // _CHUNK
    cpt = d // _NW
    flat = cpt * n
    groups = _CHUNK // _L

    mesh = plsc.VectorSubcoreMesh(core_axis_name="c", subcore_axis_name="s")

    @functools.partial(
        pl.kernel,
        out_type=jax.ShapeDtypeStruct((_NW, flat), jnp.float32),
        mesh=mesh,
        compiler_params=pltpu.CompilerParams(needs_layout_passes=False),
        scratch_types=[
            pltpu.VMEM((flat,), jnp.float32),
            pltpu.VMEM((flat,), jnp.float32),
            pltpu.VMEM((2, _CHUNK), jnp.int32),
            pltpu.VMEM((2, _CHUNK), jnp.int32),
            pltpu.VMEM((2, _CHUNK), jnp.float32),
            pltpu.SemaphoreType.DMA,
            pltpu.SemaphoreType.DMA,
        ],
    )
    def k(xt_hbm, src_hbm, dst_hbm, probs_hbm, out_hbm,
          xloc, accl, sbuf, dbuf, pbuf, gsem, isem):
        cid = lax.axis_index("c")
        sid = lax.axis_index("s")
        gid = cid * _NS + sid

        pltpu.async_copy(xt_hbm.at[gid], xloc, gsem)

        def zbody(i, carry):
            accl[pl.ds(i * _L, _L)] = jnp.zeros((_L,), jnp.float32)
            return carry
        lax.fori_loop(0, flat // _L, zbody, 0)

        pltpu.async_copy(src_hbm.at[pl.ds(0, _CHUNK)], sbuf.at[0], isem)
        pltpu.async_copy(dst_hbm.at[pl.ds(0, _CHUNK)], dbuf.at[0], isem)
        pltpu.async_copy(probs_hbm.at[pl.ds(0, _CHUNK)], pbuf.at[0], isem)
        pltpu.make_async_copy(xt_hbm.at[gid], xloc, gsem).wait()

        def chunk_body(ci, carry):
            b = ci % 2
            nb = 1 - b
            pltpu.make_async_copy(src_hbm.at[pl.ds(0, _CHUNK)],
                                  sbuf.at[0], isem).wait()
            pltpu.make_async_copy(dst_hbm.at[pl.ds(0, _CHUNK)],
                                  dbuf.at[0], isem).wait()
            pltpu.make_async_copy(probs_hbm.at[pl.ds(0, _CHUNK)],
                                  pbuf.at[0], isem).wait()
            nc = jnp.minimum(ci + 1, chunks - 1) * _CHUNK
            pltpu.async_copy(src_hbm.at[pl.ds(nc, _CHUNK)], sbuf.at[nb], isem)
            pltpu.async_copy(dst_hbm.at[pl.ds(nc, _CHUNK)], dbuf.at[nb], isem)
            pltpu.async_copy(probs_hbm.at[pl.ds(nc, _CHUNK)], pbuf.at[nb],
                             isem)

            def group_body(g, inner):
                s16 = sbuf[b, pl.ds(g * _L, _L)]
                d16 = dbuf[b, pl.ds(g * _L, _L)]
                p16 = pbuf[b, pl.ds(g * _L, _L)]
                for c in range(cpt):
                    sf = s16 + (c * n)
                    df = d16 + (c * n)
                    xv = plsc.load_gather(xloc, [sf])
                    plsc.addupdate_scatter(accl, [df], xv * p16)
                return inner
            lax.fori_loop(0, groups, group_body, 0)
            return carry
        lax.fori_loop(0, chunks, chunk_body, 0)

        pltpu.make_async_copy(src_hbm.at[pl.ds(0, _CHUNK)],
                              sbuf.at[0], isem).wait()
        pltpu.make_async_copy(dst_hbm.at[pl.ds(0, _CHUNK)],
                              dbuf.at[0], isem).wait()
        pltpu.make_async_copy(probs_hbm.at[pl.ds(0, _CHUNK)],
                              pbuf.at[0], isem).wait()

        pltpu.sync_copy(accl, out_hbm.at[gid])

    return k(xt, srcp, dstp, probsp)


def _combine(x, agg, weight, slw):
    def body(x_ref, a_ref, w_ref, s_ref, o_ref):
        s = s_ref[0, 0]
        o_ref[...] = jnp.clip(
            x_ref[...] * (1.0 + s) + a_ref[...] * w_ref[...], 0.0, 1.0)

    return pl.pallas_call(
        body,
        out_shape=jax.ShapeDtypeStruct(x.shape, x.dtype),
    )(x, agg, weight, slw)


def kernel(x, edge_index, edge_probs, weight, self_loop_weight):
    n, d = x.shape
    e = edge_index.shape[1]
    e_pad = ((e + _CHUNK - 1) // _CHUNK) * _CHUNK
    pad = e_pad - e

    src = jnp.concatenate([edge_index[0], jnp.zeros((pad,), jnp.int32)])
    dst = jnp.concatenate([edge_index[1], jnp.zeros((pad,), jnp.int32)])
    pr = jnp.concatenate(
        [edge_probs.astype(jnp.float32), jnp.zeros((pad,), jnp.float32)])

    cpt = d // _NW
    xt = x.astype(jnp.float32).T.reshape(_NW, cpt * n)

    partials = _sc_scatter(xt, src, dst, pr, n, d)
    agg = partials.reshape(d, n).T

    w2 = weight.astype(jnp.float32).reshape(1, d)
    s2 = jnp.asarray(self_loop_weight, jnp.float32).reshape(1, 1)
    return _combine(x, agg, w2, s2)


# speedup vs baseline: 2.9041x; 2.2693x over previous
import functools

import jax
import jax.numpy as jnp
from jax import lax
from jax.experimental import pallas as pl
from jax.experimental.pallas import tpu as pltpu
from jax.experimental.pallas import tpu_sc as plsc

_NC = 2
_NS = 16
_NW = _NC * _NS
_CHUNK = 4096
_L = 16


def _sc_scatter(xt, srcp, dstp, probsp, n, d):
    e_pad = srcp.shape[0]
    chunks = e_pad // _CHUNK
    cpt = d // _NW
    flat = cpt * n
    groups = _CHUNK // _L

    mesh = plsc.VectorSubcoreMesh(core_axis_name="c", subcore_axis_name="s")

    @functools.partial(
        pl.kernel,
        out_type=jax.ShapeDtypeStruct((_NW, flat), jnp.float32),
        mesh=mesh,
        compiler_params=pltpu.CompilerParams(needs_layout_passes=False),
        scratch_types=[
            pltpu.VMEM((flat,), jnp.float32),
            pltpu.VMEM((flat,), jnp.float32),
            pltpu.VMEM((2, _CHUNK), jnp.int32),
            pltpu.VMEM((2, _CHUNK), jnp.int32),
            pltpu.VMEM((2, _CHUNK), jnp.float32),
            pltpu.SemaphoreType.DMA,
            pltpu.SemaphoreType.DMA,
        ],
    )
    def k(xt_hbm, src_hbm, dst_hbm, probs_hbm, out_hbm,
          xloc, accl, sbuf, dbuf, pbuf, gsem, isem):
        cid = lax.axis_index("c")
        sid = lax.axis_index("s")
        gid = cid * _NS + sid

        pltpu.async_copy(xt_hbm.at[gid], xloc, gsem)

        def zbody(i, carry):
            accl[pl.ds(i * _L, _L)] = jnp.zeros((_L,), jnp.float32)
            return carry
        lax.fori_loop(0, flat // _L, zbody, 0)

        pltpu.async_copy(src_hbm.at[pl.ds(0, _CHUNK)], sbuf.at[0], isem)
        pltpu.async_copy(dst_hbm.at[pl.ds(0, _CHUNK)], dbuf.at[0], isem)
        pltpu.async_copy(probs_hbm.at[pl.ds(0, _CHUNK)], pbuf.at[0], isem)
        pltpu.make_async_copy(xt_hbm.at[gid], xloc, gsem).wait()

        def chunk_body(ci, carry):
            b = ci % 2
            nb = 1 - b
            pltpu.make_async_copy(src_hbm.at[pl.ds(0, _CHUNK)],
                                  sbuf.at[0], isem).wait()
            pltpu.make_async_copy(dst_hbm.at[pl.ds(0, _CHUNK)],
                                  dbuf.at[0], isem).wait()
            pltpu.make_async_copy(probs_hbm.at[pl.ds(0, _CHUNK)],
                                  pbuf.at[0], isem).wait()
            nc = jnp.minimum(ci + 1, chunks - 1) * _CHUNK
            pltpu.async_copy(src_hbm.at[pl.ds(nc, _CHUNK)], sbuf.at[nb], isem)
            pltpu.async_copy(dst_hbm.at[pl.ds(nc, _CHUNK)], dbuf.at[nb], isem)
            pltpu.async_copy(probs_hbm.at[pl.ds(nc, _CHUNK)], pbuf.at[nb],
                             isem)

            @plsc.parallel_loop(0, groups, 1, unroll=4)
            def group_body(g):
                s16 = sbuf[b, pl.ds(g * _L, _L)]
                d16 = dbuf[b, pl.ds(g * _L, _L)]
                p16 = pbuf[b, pl.ds(g * _L, _L)]
                for c in range(cpt):
                    sf = s16 + (c * n)
                    df = d16 + (c * n)
                    xv = plsc.load_gather(xloc, [sf])
                    plsc.addupdate_scatter(accl, [df], xv * p16)
            return carry
        lax.fori_loop(0, chunks, chunk_body, 0)

        pltpu.make_async_copy(src_hbm.at[pl.ds(0, _CHUNK)],
                              sbuf.at[0], isem).wait()
        pltpu.make_async_copy(dst_hbm.at[pl.ds(0, _CHUNK)],
                              dbuf.at[0], isem).wait()
        pltpu.make_async_copy(probs_hbm.at[pl.ds(0, _CHUNK)],
                              pbuf.at[0], isem).wait()

        pltpu.sync_copy(accl, out_hbm.at[gid])

    return k(xt, srcp, dstp, probsp)


def _combine(x, agg, weight, slw):
    def body(x_ref, a_ref, w_ref, s_ref, o_ref):
        s = s_ref[0, 0]
        o_ref[...] = jnp.clip(
            x_ref[...] * (1.0 + s) + a_ref[...] * w_ref[...], 0.0, 1.0)

    return pl.pallas_call(
        body,
        out_shape=jax.ShapeDtypeStruct(x.shape, x.dtype),
    )(x, agg, weight, slw)


def kernel(x, edge_index, edge_probs, weight, self_loop_weight):
    n, d = x.shape
    e = edge_index.shape[1]
    e_pad = ((e + _CHUNK - 1) // _CHUNK) * _CHUNK
    pad = e_pad - e

    src = jnp.concatenate([edge_index[0], jnp.zeros((pad,), jnp.int32)])
    dst = jnp.concatenate([edge_index[1], jnp.zeros((pad,), jnp.int32)])
    pr = jnp.concatenate(
        [edge_probs.astype(jnp.float32), jnp.zeros((pad,), jnp.float32)])

    cpt = d // _NW
    xt = x.astype(jnp.float32).T.reshape(_NW, cpt * n)

    partials = _sc_scatter(xt, src, dst, pr, n, d)
    agg = partials.reshape(d, n).T

    w2 = weight.astype(jnp.float32).reshape(1, d)
    s2 = jnp.asarray(self_loop_weight, jnp.float32).reshape(1, 1)
    return _combine(x, agg, w2, s2)
